# Initial kernel scaffold; baseline (speedup 1.0000x reference)
import jax
import jax.numpy as jnp
from jax.experimental import pallas as pl


def kernel(x, edge_index, W, b):
    raise NotImplementedError("write your pallas kernel here")


# speedup vs baseline: 25.9146x; 25.9146x over previous
import functools
import jax
import jax.numpy as jnp
from jax import lax
from jax.experimental import pallas as pl
from jax.experimental.pallas import tpu as pltpu
from jax.experimental.pallas import tpu_sc as plsc

N = 10000
E = 320000
F = 128
NP = 10240
NC, NS = 2, 16
NW = NC * NS
EPW = E // NW
K = 80
NCH = EPW // K
RPT = NP // NS

_mesh = functools.partial(
    plsc.VectorSubcoreMesh, core_axis_name="c", subcore_axis_name="s",
    num_cores=NC, num_subcores=NS)


@functools.partial(
    pl.kernel,
    out_type=jax.ShapeDtypeStruct((NC, NP), jnp.float32),
    mesh=_mesh(),
    scratch_types=[
        pltpu.VMEM((NCH, K), jnp.int32),
        pltpu.VMEM((K,), jnp.float32),
        pltpu.VMEM_SHARED((NP,), jnp.float32),
        pltpu.SemaphoreType.DMA,
    ],
)
def _sc_degree(rows_hbm, zeros_hbm, out_hbm, rowv, ones_v, hist, sem):
    c = lax.axis_index("c")
    s = lax.axis_index("s")
    pltpu.sync_copy(zeros_hbm.at[pl.ds(s * RPT, RPT)],
                    hist.at[pl.ds(s * RPT, RPT)])
    pltpu.sync_copy(rows_hbm.at[c, s], rowv)
    for i in range(K // 16):
        ones_v[pl.ds(i * 16, 16)] = jnp.ones((16,), jnp.float32)
    plsc.subcore_barrier()

    def body(j, carry):
        pltpu.sync_copy(ones_v, hist.at[rowv.at[j]], add=True)
        return carry

    lax.fori_loop(0, NCH, body, 0)
    plsc.subcore_barrier()
    pltpu.sync_copy(hist.at[pl.ds(s * RPT, RPT)],
                    out_hbm.at[c, pl.ds(s * RPT, RPT)])


@functools.partial(
    pl.kernel,
    out_type=jax.ShapeDtypeStruct((NC, NP, F), jnp.float32),
    mesh=_mesh(),
    scratch_types=[
        pltpu.VMEM((NCH, K), jnp.int32),
        pltpu.VMEM((NCH, K), jnp.int32),
        pltpu.VMEM((K, F), jnp.float32),
        pltpu.VMEM_SHARED((NP, F), jnp.float32),
        pltpu.SemaphoreType.DMA,
    ],
)
def _sc_aggregate(cols_hbm, rows_hbm, y_hbm, zeros_hbm, out_hbm,
                  colv, rowv, ybuf, agg, sem):
    c = lax.axis_index("c")
    s = lax.axis_index("s")
    pltpu.sync_copy(zeros_hbm, agg.at[pl.ds(s * RPT, RPT)])
    pltpu.sync_copy(cols_hbm.at[c, s], colv)
    pltpu.sync_copy(rows_hbm.at[c, s], rowv)
    plsc.subcore_barrier()

    def body(j, carry):
        pltpu.async_copy(y_hbm.at[colv.at[j]], ybuf, sem).wait()
        pltpu.sync_copy(ybuf, agg.at[rowv.at[j]], add=True)
        return carry

    lax.fori_loop(0, NCH, body, 0)
    plsc.subcore_barrier()
    pltpu.sync_copy(agg.at[pl.ds(s * RPT, RPT)],
                    out_hbm.at[c, pl.ds(s * RPT, RPT)])


def _tc_scale_body(h0, h1, x, y):
    d = h0[...] + h1[...] + 1.0
    dinv = lax.rsqrt(d)
    y[...] = x[...] * dinv


BN = 1024

_tc_scale = pl.pallas_call(
    _tc_scale_body,
    out_shape=jax.ShapeDtypeStruct((NP, F), jnp.float32),
    grid=(NP // BN,),
    in_specs=[
        pl.BlockSpec((BN, 1), lambda i: (i, 0)),
        pl.BlockSpec((BN, 1), lambda i: (i, 0)),
        pl.BlockSpec((BN, F), lambda i: (i, 0)),
    ],
    out_specs=pl.BlockSpec((BN, F), lambda i: (i, 0)),
)


def _tc_final_body(h0, h1, y, a0, a1, w, bias, out):
    d = h0[...] + h1[...] + 1.0
    dinv = lax.rsqrt(d)
    sagg = (a0[...] + a1[...] + y[...]) * dinv
    z = jnp.dot(sagg, w[...], preferred_element_type=jnp.float32) + bias[...]
    zn = jnp.minimum(z, 0.0)
    out[...] = jnp.where(z > 0, z, jnp.exp(zn) - 1.0)


_tc_final = pl.pallas_call(
    _tc_final_body,
    out_shape=jax.ShapeDtypeStruct((NP, F), jnp.float32),
    grid=(NP // BN,),
    in_specs=[
        pl.BlockSpec((BN, 1), lambda i: (i, 0)),
        pl.BlockSpec((BN, 1), lambda i: (i, 0)),
        pl.BlockSpec((BN, F), lambda i: (i, 0)),
        pl.BlockSpec((BN, F), lambda i: (i, 0)),
        pl.BlockSpec((BN, F), lambda i: (i, 0)),
        pl.BlockSpec((F, F), lambda i: (0, 0)),
        pl.BlockSpec((1, F), lambda i: (0, 0)),
    ],
    out_specs=pl.BlockSpec((BN, F), lambda i: (i, 0)),
)


@jax.jit
def kernel(x, edge_index, W, b):
    xp = jnp.pad(x.reshape(N, F), ((0, NP - N), (0, 0)))
    rows_r = edge_index[0].reshape(NC, NS, NCH, K)
    cols_r = edge_index[1].reshape(NC, NS, NCH, K)
    zrow = jnp.zeros((NP,), jnp.float32)
    zagg = jnp.zeros((RPT, F), jnp.float32)

    hist2 = _sc_degree(rows_r, zrow)
    h0 = hist2[0].reshape(NP, 1)
    h1 = hist2[1].reshape(NP, 1)
    y = _tc_scale(h0, h1, xp)
    agg2 = _sc_aggregate(cols_r, rows_r, y, zagg)
    out = _tc_final(h0, h1, y, agg2[0], agg2[1], W, b.reshape(1, F))
    return out[:N].reshape(1, N, F)
